# baseline (speedup 1.0000x reference)
import jax
import jax.numpy as jnp
from jax import lax
from jax.experimental import pallas as pl
from jax.experimental.pallas import tpu as pltpu
from jax.experimental.pallas import tpu_sc as plsc

LOC_EMB_SIZE = 64
N_POI_CAT = 16
N_VALS = 11
BATCH = 1024
POINT = 100000
ALPHA = 0.9
L = 16
NC, NS = 2, 16
D_PER_TILE = LOC_EMB_SIZE // (NC * NS)
NGRP = BATCH // L


def _body(x_hbm, poi_t_hbm, w_hbm, loc_t_hbm, out_t_hbm,
          x_v, row_v, idx_v, w_v, e0_v, e1_v, out_rows_v,
          sem_a, sem_b, sem_w, gath_sh):
    c = lax.axis_index("c")
    s = lax.axis_index("s")
    d0 = (c * NS + s) * D_PER_TILE

    cp_x = pltpu.async_copy(x_hbm, x_v, sem_a)
    cp_w = pltpu.async_copy(w_hbm, w_v, sem_w)
    cp_poi = pltpu.async_copy(poi_t_hbm.at[s], row_v, sem_b)
    cp_x.wait()
    cp_poi.wait()

    def _phase_a(g, carry):
        xg = x_v[pl.ds(g * L, L)]
        val = plsc.load_gather(row_v, [xg])
        idx_v[s, pl.ds(g * L, L)] = val.astype(jnp.int32)
        return carry

    lax.fori_loop(0, NGRP, _phase_a, 0, unroll=4)
    pltpu.sync_copy(idx_v.at[s], gath_sh.at[s])

    cp_u0 = pltpu.async_copy(loc_t_hbm.at[d0], row_v, sem_a)

    plsc.subcore_barrier()
    pltpu.sync_copy(gath_sh, idx_v)

    cp_w.wait()
    lane = lax.iota(jnp.int32, L)
    vlane = jnp.minimum(lane, N_VALS - 1)
    e_vec0 = jnp.exp(plsc.load_gather(w_v, [vlane, jnp.broadcast_to(d0, (L,))]))
    e_vec1 = jnp.exp(plsc.load_gather(w_v, [vlane, jnp.broadcast_to(d0 + 1, (L,))]))
    e0_v[...] = e_vec0
    e1_v[...] = e_vec1

    def _lv_pass(dd):
        def _p(g, carry):
            xg = x_v[pl.ds(g * L, L)]
            lv = plsc.load_gather(row_v, [xg])
            out_rows_v[dd, pl.ds(g * L, L)] = lv * ALPHA
            return carry
        lax.fori_loop(0, NGRP, _p, 0, unroll=4)

    cp_u0.wait()
    _lv_pass(0)
    cp_u1 = pltpu.async_copy(loc_t_hbm.at[d0 + 1], row_v, sem_b)

    def _p_pass(g, carry):
        xg = x_v[pl.ds(g * L, L)]
        ik = idx_v[0, pl.ds(g * L, L)]
        ga0 = plsc.load_gather(e0_v, [ik])
        ga1 = plsc.load_gather(e1_v, [ik])
        den0, num0 = ga0, ga0 * ga0
        den1, num1 = ga1, ga1 * ga1
        for k in range(1, N_POI_CAT):
            ik = idx_v[k, pl.ds(g * L, L)]
            ga0 = plsc.load_gather(e0_v, [ik])
            ga1 = plsc.load_gather(e1_v, [ik])
            den0 = den0 + ga0
            num0 = num0 + ga0 * ga0
            den1 = den1 + ga1
            num1 = num1 + ga1 * ga1
        o = out_rows_v[0, pl.ds(g * L, L)]
        out_rows_v[0, pl.ds(g * L, L)] = o + (num0 / den0) * (1.0 - ALPHA)
        out_rows_v[1, pl.ds(g * L, L)] = (num1 / den1) * (1.0 - ALPHA)
        return carry

    lax.fori_loop(0, NGRP, _p_pass, 0, unroll=2)

    cp_u1.wait()

    def _lv_add_pass(g, carry):
        xg = x_v[pl.ds(g * L, L)]
        lv = plsc.load_gather(row_v, [xg])
        o = out_rows_v[1, pl.ds(g * L, L)]
        out_rows_v[1, pl.ds(g * L, L)] = o + lv * ALPHA
        return carry

    lax.fori_loop(0, NGRP, _lv_add_pass, 0, unroll=4)

    pltpu.sync_copy(out_rows_v, out_t_hbm.at[pl.ds(d0, D_PER_TILE)])


@jax.jit
def kernel(x, poi, emb_poi_weight, emb_loc_weight):
    run = pl.kernel(
        _body,
        out_type=jax.ShapeDtypeStruct((LOC_EMB_SIZE, BATCH), jnp.float32),
        mesh=plsc.VectorSubcoreMesh(core_axis_name="c", subcore_axis_name="s"),
        compiler_params=pltpu.CompilerParams(needs_layout_passes=False,
                                             use_tc_tiling_on_sc=True),
        scratch_types=[
            pltpu.VMEM((BATCH,), jnp.int32),
            pltpu.VMEM((POINT,), jnp.float32),
            pltpu.VMEM((N_POI_CAT, BATCH), jnp.int32),
            pltpu.VMEM((N_VALS, LOC_EMB_SIZE), jnp.float32),
            pltpu.VMEM((L,), jnp.float32),
            pltpu.VMEM((L,), jnp.float32),
            pltpu.VMEM((D_PER_TILE, BATCH), jnp.float32),
            pltpu.SemaphoreType.DMA,
            pltpu.SemaphoreType.DMA,
            pltpu.SemaphoreType.DMA,
            pltpu.VMEM_SHARED((N_POI_CAT, BATCH), jnp.int32),
        ],
    )
    out_t = run(x.astype(jnp.int32), poi.T, emb_poi_weight, emb_loc_weight.T)
    return out_t.T
